# Initial kernel scaffold; baseline (speedup 1.0000x reference)
import functools

import jax
import jax.numpy as jnp
from jax import lax
from jax.experimental import pallas as pl
from jax.experimental.pallas import tpu as pltpu
from jax.experimental.pallas import tpu_sc as plsc

N = 10000
E = 320000
D = 128

NC = 2
NS = 16
NW = NC * NS

C = 128
CH_PER_W = 79
NCH = NW * CH_PER_W
EPAD = NCH * C
NPAD = 10240
ROWS_PER_TILE = NPAD // NS

RB = 1000


def _dotT(a, w):
    return lax.dot_general(a, w, (((1,), (1,)), ((), ())),
                           preferred_element_type=jnp.float32)


def _tc1_body(x_ref, wl_ref, wr_ref, bl_ref, y1_ref, r1_ref):
    xb = x_ref[...]
    y1_ref[...] = _dotT(xb, wl_ref[...])
    r1_ref[...] = _dotT(xb, wr_ref[...]) + bl_ref[...]


def _tc2_body(p_ref, c_ref, r1_ref, g_ref, b_ref, wl2_ref, wr2_ref, bl2_ref,
              y2_ref, r2_ref):
    p = p_ref[0] + p_ref[1]
    cnt = jnp.maximum(c_ref[0, :, 0:1] + c_ref[1, :, 0:1], 1.0)
    h = p / cnt + r1_ref[...]
    mu = jnp.mean(h, axis=1, keepdims=True)
    var = jnp.mean((h - mu) ** 2, axis=1, keepdims=True)
    hn = (h - mu) / jnp.sqrt(var + 1e-5) * g_ref[...] + b_ref[...]
    ge = 0.5 * hn * (1.0 + lax.erf(hn * 0.7071067811865476))
    y2_ref[...] = _dotT(ge, wl2_ref[...])
    r2_ref[...] = _dotT(ge, wr2_ref[...]) + bl2_ref[...]


def _tc3_body(p_ref, c_ref, r2_ref, out_ref):
    p = p_ref[0] + p_ref[1]
    cnt = jnp.maximum(c_ref[0, :, 0:1] + c_ref[1, :, 0:1], 1.0)
    o = p / cnt + r2_ref[...]
    m = jnp.max(o, axis=1, keepdims=True)
    s = jnp.sum(jnp.exp(o - m), axis=1, keepdims=True)
    out_ref[...] = (o - m) - jnp.log(s)


_row_spec = pl.BlockSpec((RB, D), lambda i: (i, 0))
_w_spec = pl.BlockSpec((D, D), lambda i: (0, 0))
_b_spec = pl.BlockSpec((1, D), lambda i: (0, 0))
_p_spec = pl.BlockSpec((2, RB, D), lambda i: (0, i, 0))
_c_spec = pl.BlockSpec((2, RB, 16), lambda i: (0, i, 0))

_tc1 = pl.pallas_call(
    _tc1_body,
    grid=(N // RB,),
    in_specs=[_row_spec, _w_spec, _w_spec, _b_spec],
    out_specs=[_row_spec, _row_spec],
    out_shape=[jax.ShapeDtypeStruct((N, D), jnp.float32)] * 2,
)

_tc2 = pl.pallas_call(
    _tc2_body,
    grid=(N // RB,),
    in_specs=[_p_spec, _c_spec, _row_spec, _b_spec, _b_spec, _w_spec, _w_spec,
              _b_spec],
    out_specs=[_row_spec, _row_spec],
    out_shape=[jax.ShapeDtypeStruct((N, D), jnp.float32)] * 2,
)

_tc3 = pl.pallas_call(
    _tc3_body,
    grid=(N // RB,),
    in_specs=[_p_spec, _c_spec, _row_spec],
    out_specs=_row_spec,
    out_shape=jax.ShapeDtypeStruct((N, D), jnp.float32),
)


def _make_sc_agg(with_counts):
    scratch = [
        pltpu.VMEM((CH_PER_W, C), jnp.int32),
        pltpu.VMEM((CH_PER_W, C), jnp.int32),
        pltpu.VMEM((C, D), jnp.float32),
        pltpu.VMEM((16, D), jnp.float32),
        pltpu.VMEM_SHARED((NPAD, D), jnp.float32),
        pltpu.SemaphoreType.DMA,
    ]
    out_type = [jax.ShapeDtypeStruct((NC * NPAD, D), jnp.float32)]
    if with_counts:
        scratch += [
            pltpu.VMEM((C, 16), jnp.float32),
            pltpu.VMEM((16, 16), jnp.float32),
            pltpu.VMEM_SHARED((NPAD, 16), jnp.float32),
        ]
        out_type.append(jax.ShapeDtypeStruct((NC * NPAD, 16), jnp.float32))

    def body(y_hbm, src_hbm, dst_hbm, *rest):
        if with_counts:
            (part_out, cnt_out, sidx, didx, rows, zrow, acc, sem,
             ones_v, zcnt, cacc) = rest
        else:
            part_out, sidx, didx, rows, zrow, acc, sem = rest
        cid = lax.axis_index("c")
        sid = lax.axis_index("s")
        wid = sid * NC + cid

        z16 = jnp.zeros((16,), jnp.float32)
        for r in range(16):
            for c in range(D // 16):
                zrow[r, pl.ds(c * 16, 16)] = z16
        if with_counts:
            o16 = jnp.ones((16,), jnp.float32)
            for r in range(C):
                ones_v[r, pl.ds(0, 16)] = o16
            for r in range(16):
                zcnt[r, pl.ds(0, 16)] = z16

        base = sid * ROWS_PER_TILE
        for t in range(ROWS_PER_TILE // 16):
            pltpu.sync_copy(zrow, acc.at[pl.ds(base + t * 16, 16)])
            if with_counts:
                pltpu.sync_copy(zcnt, cacc.at[pl.ds(base + t * 16, 16)])

        pltpu.sync_copy(src_hbm.at[pl.ds(wid * CH_PER_W, CH_PER_W)], sidx)
        pltpu.sync_copy(dst_hbm.at[pl.ds(wid * CH_PER_W, CH_PER_W)], didx)

        plsc.subcore_barrier()

        def chunk(j, carry):
            pltpu.async_copy(y_hbm.at[sidx.at[j]], rows, sem).wait()
            pltpu.sync_copy(rows, acc.at[didx.at[j]], add=True)
            if with_counts:
                pltpu.sync_copy(ones_v, cacc.at[didx.at[j]], add=True)
            return carry

        lax.fori_loop(0, CH_PER_W, chunk, 0)

        plsc.subcore_barrier()

        obase = cid * NPAD + base
        pltpu.sync_copy(acc.at[pl.ds(base, ROWS_PER_TILE)],
                        part_out.at[pl.ds(obase, ROWS_PER_TILE)])
        if with_counts:
            pltpu.sync_copy(cacc.at[pl.ds(base, ROWS_PER_TILE)],
                            cnt_out.at[pl.ds(obase, ROWS_PER_TILE)])

    mesh = plsc.VectorSubcoreMesh(core_axis_name="c", subcore_axis_name="s")
    return pl.kernel(body, out_type=out_type, mesh=mesh, scratch_types=scratch)


_sc_agg_cnt = _make_sc_agg(True)
_sc_agg = _make_sc_agg(False)


def kernel(x, edge_index, Wl1, bl1, Wr1, gamma, beta, Wl2, bl2, Wr2):
    src = edge_index[0]
    dst = edge_index[1]
    pad = EPAD - E
    src_p = jnp.concatenate([src, jnp.zeros((pad,), jnp.int32)]).reshape(NCH, C)
    dst_p = jnp.concatenate(
        [dst, jnp.full((pad,), NPAD - 1, jnp.int32)]).reshape(NCH, C)

    bl1r = bl1.reshape(1, D)
    bl2r = bl2.reshape(1, D)
    gr = gamma.reshape(1, D)
    br = beta.reshape(1, D)

    y1, r1 = _tc1(x, Wl1, Wr1, bl1r)
    part1, cnt16 = _sc_agg_cnt(y1, src_p, dst_p)
    part1 = part1.reshape(NC, NPAD, D)
    cnt16 = cnt16.reshape(NC, NPAD, 16)
    y2, r2 = _tc2(part1, cnt16, r1, gr, br, Wl2, Wr2, bl2r)
    part2 = _sc_agg(y2, src_p, dst_p)
    part2 = part2.reshape(NC, NPAD, D)
    return _tc3(part2, cnt16, r2)


# speedup vs baseline: 3.4059x; 3.4059x over previous
import functools

import jax
import jax.numpy as jnp
from jax import lax
from jax.experimental import pallas as pl
from jax.experimental.pallas import tpu as pltpu
from jax.experimental.pallas import tpu_sc as plsc

N = 10000
E = 320000
D = 128

NC = 2
NS = 16
NW = NC * NS

C = 128
CH_PER_W = 80
NCH = NW * CH_PER_W
EPAD = NCH * C
NPAD = 10240
ROWS_PER_TILE = NPAD // NS

RB = 1024


def _dotT(a, w):
    return lax.dot_general(a, w, (((1,), (1,)), ((), ())),
                           preferred_element_type=jnp.float32)


def _tc1_body(x_ref, wl_ref, wr_ref, bl_ref, y1_ref, r1_ref):
    xb = x_ref[...]
    y1_ref[...] = _dotT(xb, wl_ref[...])
    r1_ref[...] = _dotT(xb, wr_ref[...]) + bl_ref[...]


def _cnt_col(c_ref):
    i = pl.program_id(0)
    seg = c_ref[0, pl.ds(i * RB, RB)] + c_ref[1, pl.ds(i * RB, RB)]
    return jnp.maximum(seg, 1.0).reshape(RB, 1)


def _tc2_body(p_ref, c_ref, r1_ref, g_ref, b_ref, wl2_ref, wr2_ref, bl2_ref,
              y2_ref, r2_ref):
    p = p_ref[0] + p_ref[1]
    cnt = _cnt_col(c_ref)
    h = p / cnt + r1_ref[...]
    mu = jnp.mean(h, axis=1, keepdims=True)
    var = jnp.mean((h - mu) ** 2, axis=1, keepdims=True)
    hn = (h - mu) / jnp.sqrt(var + 1e-5) * g_ref[...] + b_ref[...]
    ge = 0.5 * hn * (1.0 + lax.erf(hn * 0.7071067811865476))
    y2_ref[...] = _dotT(ge, wl2_ref[...])
    r2_ref[...] = _dotT(ge, wr2_ref[...]) + bl2_ref[...]


def _tc3_body(p_ref, c_ref, r2_ref, out_ref):
    p = p_ref[0] + p_ref[1]
    cnt = _cnt_col(c_ref)
    o = p / cnt + r2_ref[...]
    m = jnp.max(o, axis=1, keepdims=True)
    s = jnp.sum(jnp.exp(o - m), axis=1, keepdims=True)
    out_ref[...] = (o - m) - jnp.log(s)


_row_spec = pl.BlockSpec((RB, D), lambda i: (i, 0))
_w_spec = pl.BlockSpec((D, D), lambda i: (0, 0))
_b_spec = pl.BlockSpec((1, D), lambda i: (0, 0))
_p_spec = pl.BlockSpec((2, RB, D), lambda i: (0, i, 0))
_c_spec = pl.BlockSpec((2, NPAD), lambda i: (0, 0))

_tc1 = pl.pallas_call(
    _tc1_body,
    grid=(NPAD // RB,),
    in_specs=[_row_spec, _w_spec, _w_spec, _b_spec],
    out_specs=[_row_spec, _row_spec],
    out_shape=[jax.ShapeDtypeStruct((NPAD, D), jnp.float32)] * 2,
)

_tc2 = pl.pallas_call(
    _tc2_body,
    grid=(NPAD // RB,),
    in_specs=[_p_spec, _c_spec, _row_spec, _b_spec, _b_spec, _w_spec, _w_spec,
              _b_spec],
    out_specs=[_row_spec, _row_spec],
    out_shape=[jax.ShapeDtypeStruct((NPAD, D), jnp.float32)] * 2,
)

_tc3 = pl.pallas_call(
    _tc3_body,
    grid=(NPAD // RB,),
    in_specs=[_p_spec, _c_spec, _row_spec],
    out_specs=_row_spec,
    out_shape=jax.ShapeDtypeStruct((NPAD, D), jnp.float32),
)


def _make_sc_agg(with_counts):
    scratch = [
        pltpu.VMEM((CH_PER_W, C), jnp.int32),
        pltpu.VMEM((CH_PER_W, C), jnp.int32),
        pltpu.VMEM((C, D), jnp.float32),
        pltpu.VMEM((16, D), jnp.float32),
        pltpu.VMEM_SHARED((NPAD, D), jnp.float32),
        pltpu.SemaphoreType.DMA,
    ]
    out_type = jax.ShapeDtypeStruct((NC * NPAD, D), jnp.float32)
    if with_counts:
        scratch += [
            pltpu.VMEM((C,), jnp.float32),
            pltpu.VMEM((ROWS_PER_TILE,), jnp.float32),
            pltpu.VMEM_SHARED((NPAD,), jnp.float32),
        ]
        out_type = [out_type, jax.ShapeDtypeStruct((NC * NPAD,), jnp.float32)]

    def body(y_hbm, src_hbm, dst_hbm, *rest):
        if with_counts:
            (part_out, cnt_out, sidx, didx, rows, zrow, acc, sem,
             ones_v, zcnt, cacc) = rest
        else:
            part_out, sidx, didx, rows, zrow, acc, sem = rest
        cid = lax.axis_index("c")
        sid = lax.axis_index("s")
        wid = sid * NC + cid

        z16 = jnp.zeros((16,), jnp.float32)
        for r in range(16):
            for c in range(D // 16):
                zrow[r, pl.ds(c * 16, 16)] = z16
        if with_counts:
            o16 = jnp.ones((16,), jnp.float32)
            for c in range(C // 16):
                ones_v[pl.ds(c * 16, 16)] = o16
            for c in range(ROWS_PER_TILE // 16):
                zcnt[pl.ds(c * 16, 16)] = z16

        base = sid * ROWS_PER_TILE
        for t in range(ROWS_PER_TILE // 16):
            pltpu.sync_copy(zrow, acc.at[pl.ds(base + t * 16, 16)])
        if with_counts:
            pltpu.sync_copy(zcnt, cacc.at[pl.ds(base, ROWS_PER_TILE)])

        pltpu.sync_copy(src_hbm.at[wid], sidx)
        pltpu.sync_copy(dst_hbm.at[wid], didx)

        plsc.subcore_barrier()

        def chunk(j, carry):
            pltpu.async_copy(y_hbm.at[sidx.at[j]], rows, sem).wait()
            pltpu.sync_copy(rows, acc.at[didx.at[j]], add=True)
            if with_counts:
                pltpu.sync_copy(ones_v, cacc.at[didx.at[j]], add=True)
            return carry

        lax.fori_loop(0, CH_PER_W, chunk, 0)

        plsc.subcore_barrier()

        obase = cid * NPAD + base
        pltpu.sync_copy(acc.at[pl.ds(base, ROWS_PER_TILE)],
                        part_out.at[pl.ds(obase, ROWS_PER_TILE)])
        if with_counts:
            pltpu.sync_copy(cacc.at[pl.ds(base, ROWS_PER_TILE)],
                            cnt_out.at[pl.ds(obase, ROWS_PER_TILE)])

    mesh = plsc.VectorSubcoreMesh(core_axis_name="c", subcore_axis_name="s")
    return pl.kernel(body, out_type=out_type, mesh=mesh, scratch_types=scratch)


@functools.lru_cache(maxsize=None)
def _sc_aggs():
    return _make_sc_agg(True), _make_sc_agg(False)


def kernel(x, edge_index, Wl1, bl1, Wr1, gamma, beta, Wl2, bl2, Wr2):
    src = edge_index[0]
    dst = edge_index[1]
    pad = EPAD - E
    src_p = jnp.concatenate(
        [src, jnp.zeros((pad,), jnp.int32)]).reshape(NW, CH_PER_W, C)
    dst_p = jnp.concatenate(
        [dst, jnp.full((pad,), NPAD - 1, jnp.int32)]).reshape(NW, CH_PER_W, C)

    bl1r = bl1.reshape(1, D)
    bl2r = bl2.reshape(1, D)
    gr = gamma.reshape(1, D)
    br = beta.reshape(1, D)

    sc_agg_cnt, sc_agg = _sc_aggs()
    xp = jnp.concatenate([x, jnp.zeros((NPAD - N, D), jnp.float32)])
    y1, r1 = _tc1(xp, Wl1, Wr1, bl1r)
    part1, cnt = sc_agg_cnt(y1, src_p, dst_p)
    part1 = part1.reshape(NC, NPAD, D)
    cnt = cnt.reshape(NC, NPAD)
    y2, r2 = _tc2(part1, cnt, r1, gr, br, Wl2, Wr2, bl2r)
    part2 = sc_agg(y2, src_p, dst_p)
    part2 = part2.reshape(NC, NPAD, D)
    return _tc3(part2, cnt, r2)[:N]


# speedup vs baseline: 3.7141x; 1.0905x over previous
import functools

import jax
import jax.numpy as jnp
from jax import lax
from jax.experimental import pallas as pl
from jax.experimental.pallas import tpu as pltpu
from jax.experimental.pallas import tpu_sc as plsc

N = 10000
E = 320000
D = 128

NC = 2
NS = 16
NW = NC * NS

C = 128
CH_PER_W = 80
NCH = NW * CH_PER_W
EPAD = NCH * C
NPAD = 10240
ROWS_PER_TILE = NPAD // NS

RB = 1024


def _dotT(a, w):
    return lax.dot_general(a, w, (((1,), (1,)), ((), ())),
                           preferred_element_type=jnp.float32)


def _tc1_body(x_ref, wl_ref, wr_ref, bl_ref, y1_ref, r1_ref):
    xb = x_ref[...]
    y1_ref[...] = _dotT(xb, wl_ref[...])
    r1_ref[...] = _dotT(xb, wr_ref[...]) + bl_ref[...]


def _cnt_col(c_ref):
    i = pl.program_id(0)
    seg = c_ref[0, pl.ds(i * RB, RB)] + c_ref[1, pl.ds(i * RB, RB)]
    return jnp.maximum(seg, 1.0).reshape(RB, 1)


def _tc2_body(p_ref, c_ref, r1_ref, g_ref, b_ref, wl2_ref, wr2_ref, bl2_ref,
              y2_ref, r2_ref):
    p = p_ref[0] + p_ref[1]
    cnt = _cnt_col(c_ref)
    h = p / cnt + r1_ref[...]
    mu = jnp.mean(h, axis=1, keepdims=True)
    var = jnp.mean((h - mu) ** 2, axis=1, keepdims=True)
    hn = (h - mu) / jnp.sqrt(var + 1e-5) * g_ref[...] + b_ref[...]
    ge = 0.5 * hn * (1.0 + lax.erf(hn * 0.7071067811865476))
    y2_ref[...] = _dotT(ge, wl2_ref[...])
    r2_ref[...] = _dotT(ge, wr2_ref[...]) + bl2_ref[...]


def _tc3_body(p_ref, c_ref, r2_ref, out_ref):
    p = p_ref[0] + p_ref[1]
    cnt = _cnt_col(c_ref)
    o = p / cnt + r2_ref[...]
    m = jnp.max(o, axis=1, keepdims=True)
    s = jnp.sum(jnp.exp(o - m), axis=1, keepdims=True)
    out_ref[...] = (o - m) - jnp.log(s)


_row_spec = pl.BlockSpec((RB, D), lambda i: (i, 0))
_w_spec = pl.BlockSpec((D, D), lambda i: (0, 0))
_b_spec = pl.BlockSpec((1, D), lambda i: (0, 0))
_p_spec = pl.BlockSpec((2, RB, D), lambda i: (0, i, 0))
_c_spec = pl.BlockSpec((2, NPAD), lambda i: (0, 0))

_tc1 = pl.pallas_call(
    _tc1_body,
    grid=(NPAD // RB,),
    in_specs=[_row_spec, _w_spec, _w_spec, _b_spec],
    out_specs=[_row_spec, _row_spec],
    out_shape=[jax.ShapeDtypeStruct((NPAD, D), jnp.float32)] * 2,
)

_tc2 = pl.pallas_call(
    _tc2_body,
    grid=(NPAD // RB,),
    in_specs=[_p_spec, _c_spec, _row_spec, _b_spec, _b_spec, _w_spec, _w_spec,
              _b_spec],
    out_specs=[_row_spec, _row_spec],
    out_shape=[jax.ShapeDtypeStruct((NPAD, D), jnp.float32)] * 2,
)

_tc3 = pl.pallas_call(
    _tc3_body,
    grid=(NPAD // RB,),
    in_specs=[_p_spec, _c_spec, _row_spec],
    out_specs=_row_spec,
    out_shape=jax.ShapeDtypeStruct((NPAD, D), jnp.float32),
)


def _make_sc_agg(with_counts):
    HCH = CH_PER_W // 2
    scratch = [
        pltpu.VMEM((HCH, C), jnp.int32),
        pltpu.VMEM((HCH, C), jnp.int32),
        pltpu.VMEM((C, D), jnp.float32),
        pltpu.VMEM((C, D), jnp.float32),
        pltpu.VMEM((16, D), jnp.float32),
        pltpu.VMEM_SHARED((NPAD, D), jnp.float32),
        pltpu.SemaphoreType.DMA,
        pltpu.SemaphoreType.DMA,
    ]
    out_type = jax.ShapeDtypeStruct((NC * NPAD, D), jnp.float32)
    if with_counts:
        scratch += [
            pltpu.VMEM((C,), jnp.float32),
            pltpu.VMEM((ROWS_PER_TILE,), jnp.float32),
            pltpu.VMEM_SHARED((NPAD,), jnp.float32),
        ]
        out_type = [out_type, jax.ShapeDtypeStruct((NC * NPAD,), jnp.float32)]

    def body(y_hbm, src_hbm, dst_hbm, *rest):
        if with_counts:
            (part_out, cnt_out, sidx, didx, rows0, rows1, zrow, acc, sem0,
             sem1, ones_v, zcnt, cacc) = rest
        else:
            part_out, sidx, didx, rows0, rows1, zrow, acc, sem0, sem1 = rest
        cid = lax.axis_index("c")
        sid = lax.axis_index("s")
        wid = sid * NC + cid

        z16 = jnp.zeros((16,), jnp.float32)
        for r in range(16):
            for c in range(D // 16):
                zrow[r, pl.ds(c * 16, 16)] = z16
        if with_counts:
            o16 = jnp.ones((16,), jnp.float32)
            for c in range(C // 16):
                ones_v[pl.ds(c * 16, 16)] = o16
            for c in range(ROWS_PER_TILE // 16):
                zcnt[pl.ds(c * 16, 16)] = z16

        base = sid * ROWS_PER_TILE
        for t in range(ROWS_PER_TILE // 16):
            pltpu.sync_copy(zrow, acc.at[pl.ds(base + t * 16, 16)])
        if with_counts:
            pltpu.sync_copy(zcnt, cacc.at[pl.ds(base, ROWS_PER_TILE)])

        plsc.subcore_barrier()

        def pair(t, carry):
            j0 = 2 * t
            j1 = 2 * t + 1
            pltpu.async_copy(y_hbm.at[sidx.at[j1]], rows1, sem1)
            pltpu.make_async_copy(y_hbm.at[sidx.at[j0]], rows0, sem0).wait()
            pltpu.sync_copy(rows0, acc.at[didx.at[j0]], add=True)
            if with_counts:
                pltpu.sync_copy(ones_v, cacc.at[didx.at[j0]], add=True)
            jn = jnp.minimum(j0 + 2, HCH - 1)
            pltpu.async_copy(y_hbm.at[sidx.at[jn]], rows0, sem0)
            pltpu.make_async_copy(y_hbm.at[sidx.at[j1]], rows1, sem1).wait()
            pltpu.sync_copy(rows1, acc.at[didx.at[j1]], add=True)
            if with_counts:
                pltpu.sync_copy(ones_v, cacc.at[didx.at[j1]], add=True)
            return carry

        for p in range(2):
            pltpu.sync_copy(src_hbm.at[wid, pl.ds(p * HCH, HCH)], sidx)
            pltpu.sync_copy(dst_hbm.at[wid, pl.ds(p * HCH, HCH)], didx)
            pltpu.async_copy(y_hbm.at[sidx.at[0]], rows0, sem0)
            lax.fori_loop(0, HCH // 2, pair, 0)
            pltpu.make_async_copy(y_hbm.at[sidx.at[0]], rows0, sem0).wait()

        plsc.subcore_barrier()

        obase = cid * NPAD + base
        pltpu.sync_copy(acc.at[pl.ds(base, ROWS_PER_TILE)],
                        part_out.at[pl.ds(obase, ROWS_PER_TILE)])
        if with_counts:
            pltpu.sync_copy(cacc.at[pl.ds(base, ROWS_PER_TILE)],
                            cnt_out.at[pl.ds(obase, ROWS_PER_TILE)])

    mesh = plsc.VectorSubcoreMesh(core_axis_name="c", subcore_axis_name="s")
    return pl.kernel(body, out_type=out_type, mesh=mesh, scratch_types=scratch)


@functools.lru_cache(maxsize=None)
def _sc_aggs():
    return _make_sc_agg(True), _make_sc_agg(False)


def kernel(x, edge_index, Wl1, bl1, Wr1, gamma, beta, Wl2, bl2, Wr2):
    src = edge_index[0]
    dst = edge_index[1]
    pad = EPAD - E
    src_p = jnp.concatenate(
        [src, jnp.zeros((pad,), jnp.int32)]).reshape(NW, CH_PER_W, C)
    dst_p = jnp.concatenate(
        [dst, jnp.full((pad,), NPAD - 1, jnp.int32)]).reshape(NW, CH_PER_W, C)

    bl1r = bl1.reshape(1, D)
    bl2r = bl2.reshape(1, D)
    gr = gamma.reshape(1, D)
    br = beta.reshape(1, D)

    sc_agg_cnt, sc_agg = _sc_aggs()
    xp = jnp.concatenate([x, jnp.zeros((NPAD - N, D), jnp.float32)])
    y1, r1 = _tc1(xp, Wl1, Wr1, bl1r)
    part1, cnt = sc_agg_cnt(y1, src_p, dst_p)
    part1 = part1.reshape(NC, NPAD, D)
    cnt = cnt.reshape(NC, NPAD)
    y2, r2 = _tc2(part1, cnt, r1, gr, br, Wl2, Wr2, bl2r)
    part2 = sc_agg(y2, src_p, dst_p)
    part2 = part2.reshape(NC, NPAD, D)
    return _tc3(part2, cnt, r2)[:N]


# speedup vs baseline: 3.8859x; 1.0463x over previous
import functools

import jax
import jax.numpy as jnp
from jax import lax
from jax.experimental import pallas as pl
from jax.experimental.pallas import tpu as pltpu
from jax.experimental.pallas import tpu_sc as plsc

N = 10000
E = 320000
D = 128

NC = 2
NS = 16
NW = NC * NS

C = 128
NCH = 2560
EPAD = NCH * C
NPAD = 10240
ROWS_PER_TILE = NPAD // NS

FAST_CID = 0
PASS_CH = 32
FAST_PASSES = 4
FAST_TOTAL = NS * FAST_PASSES * PASS_CH

RB = 1024


def _dotT(a, w):
    return lax.dot_general(a, w, (((1,), (1,)), ((), ())),
                           preferred_element_type=jnp.float32)


def _tc1_body(x_ref, wl_ref, wr_ref, bl_ref, y1_ref, r1_ref):
    xb = x_ref[...]
    y1_ref[...] = _dotT(xb, wl_ref[...])
    r1_ref[...] = _dotT(xb, wr_ref[...]) + bl_ref[...]


def _cnt_col(c_ref):
    i = pl.program_id(0)
    seg = c_ref[0, pl.ds(i * RB, RB)] + c_ref[1, pl.ds(i * RB, RB)]
    return jnp.maximum(seg, 1.0).reshape(RB, 1)


def _tc2_body(p_ref, c_ref, r1_ref, g_ref, b_ref, wl2_ref, wr2_ref, bl2_ref,
              y2_ref, r2_ref):
    p = p_ref[0] + p_ref[1]
    cnt = _cnt_col(c_ref)
    h = p / cnt + r1_ref[...]
    mu = jnp.mean(h, axis=1, keepdims=True)
    var = jnp.mean((h - mu) ** 2, axis=1, keepdims=True)
    hn = (h - mu) / jnp.sqrt(var + 1e-5) * g_ref[...] + b_ref[...]
    ge = 0.5 * hn * (1.0 + lax.erf(hn * 0.7071067811865476))
    y2_ref[...] = _dotT(ge, wl2_ref[...])
    r2_ref[...] = _dotT(ge, wr2_ref[...]) + bl2_ref[...]


def _tc3_body(p_ref, c_ref, r2_ref, out_ref):
    p = p_ref[0] + p_ref[1]
    cnt = _cnt_col(c_ref)
    o = p / cnt + r2_ref[...]
    m = jnp.max(o, axis=1, keepdims=True)
    s = jnp.sum(jnp.exp(o - m), axis=1, keepdims=True)
    out_ref[...] = (o - m) - jnp.log(s)


_row_spec = pl.BlockSpec((RB, D), lambda i: (i, 0))
_w_spec = pl.BlockSpec((D, D), lambda i: (0, 0))
_b_spec = pl.BlockSpec((1, D), lambda i: (0, 0))
_p_spec = pl.BlockSpec((2, RB, D), lambda i: (0, i, 0))
_c_spec = pl.BlockSpec((2, NPAD), lambda i: (0, 0))

_tc1 = pl.pallas_call(
    _tc1_body,
    grid=(NPAD // RB,),
    in_specs=[_row_spec, _w_spec, _w_spec, _b_spec],
    out_specs=[_row_spec, _row_spec],
    out_shape=[jax.ShapeDtypeStruct((NPAD, D), jnp.float32)] * 2,
)

_tc2 = pl.pallas_call(
    _tc2_body,
    grid=(NPAD // RB,),
    in_specs=[_p_spec, _c_spec, _row_spec, _b_spec, _b_spec, _w_spec, _w_spec,
              _b_spec],
    out_specs=[_row_spec, _row_spec],
    out_shape=[jax.ShapeDtypeStruct((NPAD, D), jnp.float32)] * 2,
)

_tc3 = pl.pallas_call(
    _tc3_body,
    grid=(NPAD // RB,),
    in_specs=[_p_spec, _c_spec, _row_spec],
    out_specs=_row_spec,
    out_shape=jax.ShapeDtypeStruct((NPAD, D), jnp.float32),
)


def _make_sc_agg(with_counts):
    scratch = [
        pltpu.VMEM((PASS_CH, C), jnp.int32),
        pltpu.VMEM((PASS_CH, C), jnp.int32),
        pltpu.VMEM((C, D), jnp.float32),
        pltpu.VMEM((C, D), jnp.float32),
        pltpu.VMEM((16, D), jnp.float32),
        pltpu.VMEM_SHARED((NPAD, D), jnp.float32),
        pltpu.SemaphoreType.DMA,
        pltpu.SemaphoreType.DMA,
    ]
    out_type = jax.ShapeDtypeStruct((NC * NPAD, D), jnp.float32)
    if with_counts:
        scratch += [
            pltpu.VMEM((C,), jnp.float32),
            pltpu.VMEM((ROWS_PER_TILE,), jnp.float32),
            pltpu.VMEM_SHARED((NPAD,), jnp.float32),
        ]
        out_type = [out_type, jax.ShapeDtypeStruct((NC * NPAD,), jnp.float32)]

    def body(y_hbm, src_hbm, dst_hbm, *rest):
        if with_counts:
            (part_out, cnt_out, sidx, didx, rows0, rows1, zrow, acc, sem0,
             sem1, ones_v, zcnt, cacc) = rest
        else:
            part_out, sidx, didx, rows0, rows1, zrow, acc, sem0, sem1 = rest
        cid = lax.axis_index("c")
        sid = lax.axis_index("s")
        fast = cid == FAST_CID

        z16 = jnp.zeros((16,), jnp.float32)
        for r in range(16):
            for c in range(D // 16):
                zrow[r, pl.ds(c * 16, 16)] = z16
        if with_counts:
            o16 = jnp.ones((16,), jnp.float32)
            for c in range(C // 16):
                ones_v[pl.ds(c * 16, 16)] = o16
            for c in range(ROWS_PER_TILE // 16):
                zcnt[pl.ds(c * 16, 16)] = z16

        base = sid * ROWS_PER_TILE
        for t in range(ROWS_PER_TILE // 16):
            pltpu.sync_copy(zrow, acc.at[pl.ds(base + t * 16, 16)])
        if with_counts:
            pltpu.sync_copy(zcnt, cacc.at[pl.ds(base, ROWS_PER_TILE)])

        plsc.subcore_barrier()

        def pair(t, carry):
            j0 = 2 * t
            j1 = 2 * t + 1
            pltpu.async_copy(y_hbm.at[sidx.at[j1]], rows1, sem1)
            pltpu.make_async_copy(y_hbm.at[sidx.at[j0]], rows0, sem0).wait()
            pltpu.sync_copy(rows0, acc.at[didx.at[j0]], add=True)
            if with_counts:
                pltpu.sync_copy(ones_v, cacc.at[didx.at[j0]], add=True)
            jn = jnp.minimum(j0 + 2, PASS_CH - 1)
            pltpu.async_copy(y_hbm.at[sidx.at[jn]], rows0, sem0)
            pltpu.make_async_copy(y_hbm.at[sidx.at[j1]], rows1, sem1).wait()
            pltpu.sync_copy(rows1, acc.at[didx.at[j1]], add=True)
            if with_counts:
                pltpu.sync_copy(ones_v, cacc.at[didx.at[j1]], add=True)
            return carry

        def do_pass(p, carry):
            base = jnp.where(fast, sid * (FAST_PASSES * PASS_CH) + p * PASS_CH,
                             FAST_TOTAL + sid * PASS_CH)
            base = pl.multiple_of(base, PASS_CH)
            pltpu.async_copy(src_hbm.at[pl.ds(base, PASS_CH)], sidx, sem0).wait()
            pltpu.async_copy(dst_hbm.at[pl.ds(base, PASS_CH)], didx, sem1).wait()
            pltpu.async_copy(y_hbm.at[sidx.at[0]], rows0, sem0)
            lax.fori_loop(0, PASS_CH // 2, pair, 0)
            pltpu.make_async_copy(y_hbm.at[sidx.at[0]], rows0, sem0).wait()
            return carry

        npass = jnp.where(fast, FAST_PASSES, 1)
        lax.fori_loop(0, npass, do_pass, 0)

        plsc.subcore_barrier()

        obase = cid * NPAD + base
        pltpu.sync_copy(acc.at[pl.ds(base, ROWS_PER_TILE)],
                        part_out.at[pl.ds(obase, ROWS_PER_TILE)])
        if with_counts:
            pltpu.sync_copy(cacc.at[pl.ds(base, ROWS_PER_TILE)],
                            cnt_out.at[pl.ds(obase, ROWS_PER_TILE)])

    mesh = plsc.VectorSubcoreMesh(core_axis_name="c", subcore_axis_name="s")
    return pl.kernel(body, out_type=out_type, mesh=mesh, scratch_types=scratch)


@functools.lru_cache(maxsize=None)
def _sc_aggs():
    return _make_sc_agg(True), _make_sc_agg(False)


def kernel(x, edge_index, Wl1, bl1, Wr1, gamma, beta, Wl2, bl2, Wr2):
    src = edge_index[0]
    dst = edge_index[1]
    pad = EPAD - E
    src_p = jnp.concatenate(
        [src, jnp.zeros((pad,), jnp.int32)]).reshape(NCH, C)
    dst_p = jnp.concatenate(
        [dst, jnp.full((pad,), NPAD - 1, jnp.int32)]).reshape(NCH, C)

    bl1r = bl1.reshape(1, D)
    bl2r = bl2.reshape(1, D)
    gr = gamma.reshape(1, D)
    br = beta.reshape(1, D)

    sc_agg_cnt, sc_agg = _sc_aggs()
    xp = jnp.concatenate([x, jnp.zeros((NPAD - N, D), jnp.float32)])
    y1, r1 = _tc1(xp, Wl1, Wr1, bl1r)
    part1, cnt = sc_agg_cnt(y1, src_p, dst_p)
    part1 = part1.reshape(NC, NPAD, D)
    cnt = cnt.reshape(NC, NPAD)
    y2, r2 = _tc2(part1, cnt, r1, gr, br, Wl2, Wr2, bl2r)
    part2 = sc_agg(y2, src_p, dst_p)
    part2 = part2.reshape(NC, NPAD, D)
    return _tc3(part2, cnt, r2)[:N]


# speedup vs baseline: 4.0048x; 1.0306x over previous
import functools

import jax
import jax.numpy as jnp
from jax import lax
from jax.experimental import pallas as pl
from jax.experimental.pallas import tpu as pltpu
from jax.experimental.pallas import tpu_sc as plsc

N = 10000
E = 320000
D = 128

NC = 2
NS = 16
NW = NC * NS

C = 128
NCH = 2560
EPAD = NCH * C
NPAD = 10240
ROWS_PER_TILE = NPAD // NS

PASS_CH = 40
PASSES = 2

RB = 1024


def _dotT(a, w):
    return lax.dot_general(a, w, (((1,), (1,)), ((), ())),
                           preferred_element_type=jnp.float32)


def _tc1_body(x_ref, wl_ref, wr_ref, bl_ref, y1_ref, r1_ref):
    xb = x_ref[...]
    y1_ref[...] = _dotT(xb, wl_ref[...])
    r1_ref[...] = _dotT(xb, wr_ref[...]) + bl_ref[...]


def _cnt_col(c_ref):
    i = pl.program_id(0)
    seg = c_ref[0, pl.ds(i * RB, RB)] + c_ref[1, pl.ds(i * RB, RB)]
    return jnp.maximum(seg, 1.0).reshape(RB, 1)


def _tc2_body(p_ref, c_ref, r1_ref, g_ref, b_ref, wl2_ref, wr2_ref, bl2_ref,
              y2_ref, r2_ref):
    p = p_ref[0] + p_ref[1]
    cnt = _cnt_col(c_ref)
    h = p / cnt + r1_ref[...]
    mu = jnp.mean(h, axis=1, keepdims=True)
    var = jnp.mean((h - mu) ** 2, axis=1, keepdims=True)
    hn = (h - mu) / jnp.sqrt(var + 1e-5) * g_ref[...] + b_ref[...]
    ge = 0.5 * hn * (1.0 + lax.erf(hn * 0.7071067811865476))
    y2_ref[...] = _dotT(ge, wl2_ref[...])
    r2_ref[...] = _dotT(ge, wr2_ref[...]) + bl2_ref[...]


def _tc3_body(p_ref, c_ref, r2_ref, out_ref):
    p = p_ref[0] + p_ref[1]
    cnt = _cnt_col(c_ref)
    o = p / cnt + r2_ref[...]
    m = jnp.max(o, axis=1, keepdims=True)
    s = jnp.sum(jnp.exp(o - m), axis=1, keepdims=True)
    out_ref[...] = (o - m) - jnp.log(s)


_row_spec = pl.BlockSpec((RB, D), lambda i: (i, 0))
_w_spec = pl.BlockSpec((D, D), lambda i: (0, 0))
_b_spec = pl.BlockSpec((1, D), lambda i: (0, 0))
_p_spec = pl.BlockSpec((2, RB, D), lambda i: (0, i, 0))
_c_spec = pl.BlockSpec((2, NPAD), lambda i: (0, 0))

_tc1 = pl.pallas_call(
    _tc1_body,
    grid=(NPAD // RB,),
    in_specs=[_row_spec, _w_spec, _w_spec, _b_spec],
    out_specs=[_row_spec, _row_spec],
    out_shape=[jax.ShapeDtypeStruct((NPAD, D), jnp.float32)] * 2,
)

_tc2 = pl.pallas_call(
    _tc2_body,
    grid=(NPAD // RB,),
    in_specs=[_p_spec, _c_spec, _row_spec, _b_spec, _b_spec, _w_spec, _w_spec,
              _b_spec],
    out_specs=[_row_spec, _row_spec],
    out_shape=[jax.ShapeDtypeStruct((NPAD, D), jnp.float32)] * 2,
)

_tc3 = pl.pallas_call(
    _tc3_body,
    grid=(NPAD // RB,),
    in_specs=[_p_spec, _c_spec, _row_spec],
    out_specs=_row_spec,
    out_shape=jax.ShapeDtypeStruct((NPAD, D), jnp.float32),
)


def _make_sc_agg(with_counts):
    scratch = [
        pltpu.VMEM((PASS_CH, C), jnp.int32),
        pltpu.VMEM((PASS_CH, C), jnp.int32),
        pltpu.VMEM((C, D), jnp.float32),
        pltpu.VMEM((C, D), jnp.float32),
        pltpu.VMEM((16, D), jnp.float32),
        pltpu.VMEM_SHARED((NPAD, D), jnp.float32),
        pltpu.SemaphoreType.DMA,
        pltpu.SemaphoreType.DMA,
    ]
    out_type = jax.ShapeDtypeStruct((NC * NPAD, D), jnp.float32)
    if with_counts:
        scratch += [
            pltpu.VMEM((C,), jnp.float32),
            pltpu.VMEM((ROWS_PER_TILE,), jnp.float32),
            pltpu.VMEM_SHARED((NPAD,), jnp.float32),
        ]
        out_type = [out_type, jax.ShapeDtypeStruct((NC * NPAD,), jnp.float32)]

    def body(y_hbm, src_hbm, dst_hbm, *rest):
        if with_counts:
            (part_out, cnt_out, sidx, didx, rows0, rows1, zrow, acc, sem0,
             sem1, ones_v, zcnt, cacc) = rest
        else:
            part_out, sidx, didx, rows0, rows1, zrow, acc, sem0, sem1 = rest
        cid = lax.axis_index("c")
        sid = lax.axis_index("s")
        wid = sid * NC + cid

        z16 = jnp.zeros((16,), jnp.float32)
        for r in range(16):
            for c in range(D // 16):
                zrow[r, pl.ds(c * 16, 16)] = z16
        if with_counts:
            o16 = jnp.ones((16,), jnp.float32)
            for c in range(C // 16):
                ones_v[pl.ds(c * 16, 16)] = o16
            for c in range(ROWS_PER_TILE // 16):
                zcnt[pl.ds(c * 16, 16)] = z16

        base = sid * ROWS_PER_TILE
        for t in range(ROWS_PER_TILE // 16):
            pltpu.sync_copy(zrow, acc.at[pl.ds(base + t * 16, 16)])
        if with_counts:
            pltpu.sync_copy(zcnt, cacc.at[pl.ds(base, ROWS_PER_TILE)])

        plsc.subcore_barrier()

        def pair(t, carry):
            j0 = 2 * t
            j1 = 2 * t + 1
            pltpu.async_copy(y_hbm.at[sidx.at[j1]], rows1, sem1)
            pltpu.make_async_copy(y_hbm.at[sidx.at[j0]], rows0, sem0).wait()
            pltpu.sync_copy(rows0, acc.at[didx.at[j0]], add=True)
            if with_counts:
                pltpu.sync_copy(ones_v, cacc.at[didx.at[j0]], add=True)
            jn = jnp.minimum(j0 + 2, PASS_CH - 1)
            pltpu.async_copy(y_hbm.at[sidx.at[jn]], rows0, sem0)
            pltpu.make_async_copy(y_hbm.at[sidx.at[j1]], rows1, sem1).wait()
            pltpu.sync_copy(rows1, acc.at[didx.at[j1]], add=True)
            if with_counts:
                pltpu.sync_copy(ones_v, cacc.at[didx.at[j1]], add=True)
            return carry

        def do_pass(p, carry):
            base = pl.multiple_of(wid * (PASSES * PASS_CH) + p * PASS_CH, 8)
            pltpu.async_copy(src_hbm.at[pl.ds(base, PASS_CH)], sidx, sem0).wait()
            pltpu.async_copy(dst_hbm.at[pl.ds(base, PASS_CH)], didx, sem1).wait()
            pltpu.async_copy(y_hbm.at[sidx.at[0]], rows0, sem0)
            lax.fori_loop(0, PASS_CH // 2, pair, 0)
            pltpu.make_async_copy(y_hbm.at[sidx.at[0]], rows0, sem0).wait()
            return carry

        lax.fori_loop(0, PASSES, do_pass, 0)

        plsc.subcore_barrier()

        obase = cid * NPAD + base
        pltpu.sync_copy(acc.at[pl.ds(base, ROWS_PER_TILE)],
                        part_out.at[pl.ds(obase, ROWS_PER_TILE)])
        if with_counts:
            pltpu.sync_copy(cacc.at[pl.ds(base, ROWS_PER_TILE)],
                            cnt_out.at[pl.ds(obase, ROWS_PER_TILE)])

    mesh = plsc.VectorSubcoreMesh(core_axis_name="c", subcore_axis_name="s")
    return pl.kernel(body, out_type=out_type, mesh=mesh, scratch_types=scratch)


@functools.lru_cache(maxsize=None)
def _sc_aggs():
    return _make_sc_agg(True), _make_sc_agg(False)


def kernel(x, edge_index, Wl1, bl1, Wr1, gamma, beta, Wl2, bl2, Wr2):
    src = edge_index[0]
    dst = edge_index[1]
    pad = EPAD - E
    src_p = jnp.concatenate(
        [src, jnp.zeros((pad,), jnp.int32)]).reshape(NCH, C)
    pad_dst = N + jnp.arange(pad, dtype=jnp.int32) % (NPAD - N)
    dst_p = jnp.concatenate([dst, pad_dst]).reshape(NCH, C)

    bl1r = bl1.reshape(1, D)
    bl2r = bl2.reshape(1, D)
    gr = gamma.reshape(1, D)
    br = beta.reshape(1, D)

    sc_agg_cnt, sc_agg = _sc_aggs()
    xp = jnp.concatenate([x, jnp.zeros((NPAD - N, D), jnp.float32)])
    y1, r1 = _tc1(xp, Wl1, Wr1, bl1r)
    part1, cnt = sc_agg_cnt(y1, src_p, dst_p)
    part1 = part1.reshape(NC, NPAD, D)
    cnt = cnt.reshape(NC, NPAD)
    y2, r2 = _tc2(part1, cnt, r1, gr, br, Wl2, Wr2, bl2r)
    part2 = sc_agg(y2, src_p, dst_p)
    part2 = part2.reshape(NC, NPAD, D)
    return _tc3(part2, cnt, r2)[:N]
